# baseline (speedup 1.0000x reference)
import jax
import jax.numpy as jnp
from jax.experimental import pallas as pl
from jax.experimental.pallas import tpu as pltpu

NUM_CLASSES = 81
EPS = 1e-3

N = 5000
P = 7
C = 256
H = 1024

BM = 1000
NM = N // BM
NK = P


def _head_kernel(x_ref, w1_ref, s1_ref, t1_ref, w2_ref, s2_ref, t2_ref,
                 wc_ref, bc_ref, wo_ref, bo_ref,
                 logit_ref, prob_ref, off_ref, acc_ref):
    k = pl.program_id(1)

    @pl.when(k == 0)
    def _zero():
        acc_ref[...] = jnp.zeros_like(acc_ref)

    acc = acc_ref[...]
    for w in range(P):
        xw = x_ref[:, 0, w, :].astype(jnp.bfloat16)
        ww = w1_ref[0, w, :, :].astype(jnp.bfloat16)
        acc += jnp.dot(xw, ww, preferred_element_type=jnp.float32)
    acc_ref[...] = acc

    @pl.when(k == NK - 1)
    def _epilogue():
        y1 = jnp.maximum(acc_ref[...] * s1_ref[...] + t1_ref[...], 0.0)
        y2 = jnp.dot(y1.astype(jnp.bfloat16),
                     w2_ref[0, 0].astype(jnp.bfloat16),
                     preferred_element_type=jnp.float32)
        y2 = jnp.maximum(y2 * s2_ref[...] + t2_ref[...], 0.0)
        y2b = y2.astype(jnp.bfloat16)
        logits = jnp.dot(y2b, wc_ref[...].astype(jnp.bfloat16),
                         preferred_element_type=jnp.float32) + bc_ref[...]
        logit_ref[...] = logits
        m = jnp.max(logits, axis=-1, keepdims=True)
        e = jnp.exp(logits - m)
        prob_ref[...] = e / jnp.sum(e, axis=-1, keepdims=True)
        off_ref[...] = jnp.dot(y2b, wo_ref[...].astype(jnp.bfloat16),
                               preferred_element_type=jnp.float32) + bo_ref[...]


def kernel(inputs, W1, b1, g1, be1, m1, v1, W2, b2, g2, be2, m2, v2, Wc, bc, Wo, bo):
    s1 = g1 * jax.lax.rsqrt(v1 + EPS)
    t1 = s1 * (b1 - m1) + be1
    s2 = g2 * jax.lax.rsqrt(v2 + EPS)
    t2 = s2 * (b2 - m2) + be2

    const = lambda bs: pl.BlockSpec(bs, lambda m, k: (0,) * len(bs))

    logit, prob, off = pl.pallas_call(
        _head_kernel,
        grid=(NM, NK),
        in_specs=[
            pl.BlockSpec((BM, 1, P, C), lambda m, k: (m, k, 0, 0)),
            pl.BlockSpec((1, P, C, H), lambda m, k: (k, 0, 0, 0)),
            const((1, H)), const((1, H)),
            const((1, 1, H, H)),
            const((1, H)), const((1, H)),
            const((H, NUM_CLASSES)), const((1, NUM_CLASSES)),
            const((H, 4 * NUM_CLASSES)), const((1, 4 * NUM_CLASSES)),
        ],
        out_specs=[
            pl.BlockSpec((BM, NUM_CLASSES), lambda m, k: (m, 0)),
            pl.BlockSpec((BM, NUM_CLASSES), lambda m, k: (m, 0)),
            pl.BlockSpec((BM, 4 * NUM_CLASSES), lambda m, k: (m, 0)),
        ],
        out_shape=[
            jax.ShapeDtypeStruct((N, NUM_CLASSES), jnp.float32),
            jax.ShapeDtypeStruct((N, NUM_CLASSES), jnp.float32),
            jax.ShapeDtypeStruct((N, 4 * NUM_CLASSES), jnp.float32),
        ],
        scratch_shapes=[pltpu.VMEM((BM, H), jnp.float32)],
        compiler_params=pltpu.CompilerParams(
            dimension_semantics=("parallel", "arbitrary"),
        ),
    )(inputs, W1,
      s1.reshape(1, H), t1.reshape(1, H),
      W2,
      s2.reshape(1, H), t2.reshape(1, H),
      Wc, bc.reshape(1, NUM_CLASSES),
      Wo, bo.reshape(1, 4 * NUM_CLASSES))

    return logit, prob, off.reshape(N, NUM_CLASSES, 4)


# speedup vs baseline: 1.4926x; 1.4926x over previous
import jax
import jax.numpy as jnp
from jax.experimental import pallas as pl
from jax.experimental.pallas import tpu as pltpu

NUM_CLASSES = 81
EPS = 1e-3

N = 5000
P = 7
C = 256
H = 1024
NK = P * P

BM = 1000
NM = N // BM


def _head_kernel(x_ref, w1_ref, s1_ref, t1_ref, w2_ref, s2_ref, t2_ref,
                 wc_ref, bc_ref, wo_ref, bo_ref,
                 logit_ref, prob_ref, off_ref, acc_ref):
    k = pl.program_id(0)
    m = pl.program_id(1)
    rows = pl.ds(m * BM, BM)

    @pl.when(k == 0)
    def _zero():
        acc_ref[rows, :] = jnp.zeros((BM, H), jnp.float32)

    acc_ref[rows, :] += jnp.dot(x_ref[...].astype(jnp.bfloat16),
                                w1_ref[...].astype(jnp.bfloat16),
                                preferred_element_type=jnp.float32)

    @pl.when(k == NK - 1)
    def _epilogue():
        y1 = jnp.maximum(acc_ref[rows, :] * s1_ref[...] + t1_ref[...], 0.0)
        y2 = jnp.dot(y1.astype(jnp.bfloat16),
                     w2_ref[...].astype(jnp.bfloat16),
                     preferred_element_type=jnp.float32)
        y2 = jnp.maximum(y2 * s2_ref[...] + t2_ref[...], 0.0)
        y2b = y2.astype(jnp.bfloat16)
        logits = jnp.dot(y2b, wc_ref[...].astype(jnp.bfloat16),
                         preferred_element_type=jnp.float32) + bc_ref[...]
        logit_ref[...] = logits
        mx = jnp.max(logits, axis=-1, keepdims=True)
        e = jnp.exp(logits - mx)
        prob_ref[...] = e / jnp.sum(e, axis=-1, keepdims=True)
        off_ref[...] = jnp.dot(y2b, wo_ref[...].astype(jnp.bfloat16),
                               preferred_element_type=jnp.float32) + bo_ref[...]


def kernel(inputs, W1, b1, g1, be1, m1, v1, W2, b2, g2, be2, m2, v2, Wc, bc, Wo, bo):
    xt = inputs.transpose(1, 2, 0, 3)

    s1 = g1 * jax.lax.rsqrt(v1 + EPS)
    t1 = s1 * (b1 - m1) + be1
    s2 = g2 * jax.lax.rsqrt(v2 + EPS)
    t2 = s2 * (b2 - m2) + be2

    const = lambda bs: pl.BlockSpec(bs, lambda k, m: (0,) * len(bs))

    logit, prob, off = pl.pallas_call(
        _head_kernel,
        grid=(NK, NM),
        in_specs=[
            pl.BlockSpec((None, None, BM, C), lambda k, m: (k // P, k % P, m, 0)),
            pl.BlockSpec((None, None, C, H), lambda k, m: (k // P, k % P, 0, 0)),
            const((1, H)), const((1, H)),
            pl.BlockSpec((None, None, H, H), lambda k, m: (0, 0, 0, 0)),
            const((1, H)), const((1, H)),
            const((H, NUM_CLASSES)), const((1, NUM_CLASSES)),
            const((H, 4 * NUM_CLASSES)), const((1, 4 * NUM_CLASSES)),
        ],
        out_specs=[
            pl.BlockSpec((BM, NUM_CLASSES), lambda k, m: (m, 0)),
            pl.BlockSpec((BM, NUM_CLASSES), lambda k, m: (m, 0)),
            pl.BlockSpec((BM, 4 * NUM_CLASSES), lambda k, m: (m, 0)),
        ],
        out_shape=[
            jax.ShapeDtypeStruct((N, NUM_CLASSES), jnp.float32),
            jax.ShapeDtypeStruct((N, NUM_CLASSES), jnp.float32),
            jax.ShapeDtypeStruct((N, 4 * NUM_CLASSES), jnp.float32),
        ],
        scratch_shapes=[pltpu.VMEM((N, H), jnp.float32)],
        compiler_params=pltpu.CompilerParams(
            dimension_semantics=("arbitrary", "arbitrary"),
        ),
    )(xt, W1,
      s1.reshape(1, H), t1.reshape(1, H),
      W2,
      s2.reshape(1, H), t2.reshape(1, H),
      Wc, bc.reshape(1, NUM_CLASSES),
      Wo, bo.reshape(1, 4 * NUM_CLASSES))

    return logit, prob, off.reshape(N, NUM_CLASSES, 4)


# speedup vs baseline: 2.8332x; 1.8982x over previous
import jax
import jax.numpy as jnp
from jax.experimental import pallas as pl
from jax.experimental.pallas import tpu as pltpu

NUM_CLASSES = 81
EPS = 1e-3

N = 5000
P = 7
C = 256
H = 1024

BM = 1000
NM = N // BM


def _head_kernel(x_ref, w1_ref, s1_ref, t1_ref, w2_ref, s2_ref, t2_ref,
                 wc_ref, bc_ref, wo_ref, bo_ref,
                 logit_ref, prob_ref, off_ref, acc_ref):
    k = pl.program_id(1)

    psum = jnp.dot(x_ref[0].astype(jnp.bfloat16),
                   w1_ref[0].astype(jnp.bfloat16),
                   preferred_element_type=jnp.float32)
    for w in range(1, P):
        psum += jnp.dot(x_ref[w].astype(jnp.bfloat16),
                        w1_ref[w].astype(jnp.bfloat16),
                        preferred_element_type=jnp.float32)

    @pl.when(k == 0)
    def _init():
        acc_ref[...] = psum

    @pl.when(k > 0)
    def _accum():
        acc_ref[...] += psum

    @pl.when(k == P - 1)
    def _epilogue():
        y1 = jnp.maximum(acc_ref[...] * s1_ref[...] + t1_ref[...], 0.0)
        y2 = jnp.dot(y1.astype(jnp.bfloat16),
                     w2_ref[...].astype(jnp.bfloat16),
                     preferred_element_type=jnp.float32)
        y2 = jnp.maximum(y2 * s2_ref[...] + t2_ref[...], 0.0)
        y2b = y2.astype(jnp.bfloat16)
        logits = jnp.dot(y2b, wc_ref[...].astype(jnp.bfloat16),
                         preferred_element_type=jnp.float32) + bc_ref[...]
        logit_ref[...] = logits
        mx = jnp.max(logits, axis=-1, keepdims=True)
        e = jnp.exp(logits - mx)
        prob_ref[...] = e / jnp.sum(e, axis=-1, keepdims=True)
        off_ref[...] = jnp.dot(y2b, wo_ref[...].astype(jnp.bfloat16),
                               preferred_element_type=jnp.float32) + bo_ref[...]


def kernel(inputs, W1, b1, g1, be1, m1, v1, W2, b2, g2, be2, m2, v2, Wc, bc, Wo, bo):
    xt = inputs.transpose(1, 2, 0, 3)

    s1 = g1 * jax.lax.rsqrt(v1 + EPS)
    t1 = s1 * (b1 - m1) + be1
    s2 = g2 * jax.lax.rsqrt(v2 + EPS)
    t2 = s2 * (b2 - m2) + be2

    const = lambda bs: pl.BlockSpec(bs, lambda m, k: (0,) * len(bs))

    logit, prob, off = pl.pallas_call(
        _head_kernel,
        grid=(NM, P),
        in_specs=[
            pl.BlockSpec((None, P, BM, C), lambda m, k: (k, 0, m, 0)),
            pl.BlockSpec((None, P, C, H), lambda m, k: (k, 0, 0, 0)),
            const((1, H)), const((1, H)),
            pl.BlockSpec((None, None, H, H), lambda m, k: (0, 0, 0, 0)),
            const((1, H)), const((1, H)),
            const((H, NUM_CLASSES)), const((1, NUM_CLASSES)),
            const((H, 4 * NUM_CLASSES)), const((1, 4 * NUM_CLASSES)),
        ],
        out_specs=[
            pl.BlockSpec((BM, NUM_CLASSES), lambda m, k: (m, 0)),
            pl.BlockSpec((BM, NUM_CLASSES), lambda m, k: (m, 0)),
            pl.BlockSpec((BM, 4 * NUM_CLASSES), lambda m, k: (m, 0)),
        ],
        out_shape=[
            jax.ShapeDtypeStruct((N, NUM_CLASSES), jnp.float32),
            jax.ShapeDtypeStruct((N, NUM_CLASSES), jnp.float32),
            jax.ShapeDtypeStruct((N, 4 * NUM_CLASSES), jnp.float32),
        ],
        scratch_shapes=[pltpu.VMEM((BM, H), jnp.float32)],
        compiler_params=pltpu.CompilerParams(
            dimension_semantics=("arbitrary", "arbitrary"),
        ),
    )(xt, W1,
      s1.reshape(1, H), t1.reshape(1, H),
      W2,
      s2.reshape(1, H), t2.reshape(1, H),
      Wc, bc.reshape(1, NUM_CLASSES),
      Wo, bo.reshape(1, 4 * NUM_CLASSES))

    return logit, prob, off.reshape(N, NUM_CLASSES, 4)
